# baseline (speedup 1.0000x reference)
import functools

import jax
import jax.numpy as jnp
from jax import lax
from jax.experimental import pallas as pl
from jax.experimental.pallas import tpu as pltpu
from jax.experimental.pallas import tpu_sc as plsc

N = 10000
E = 320000
H = 128
NC, NS = 2, 16
NW = NC * NS
NP = 10240
EW = E // NW
CH = 80
NCHUNK = EW // CH
NPAIR = (NCHUNK - 1) // 2
SUP = 5
SUPW = SUP * CH
NSUP = EW // SUPW
NSPAIR = (NSUP - 1) // 2
GRP = CH // 16
RPT = NP // NS

_mesh = plsc.VectorSubcoreMesh(
    core_axis_name="c", subcore_axis_name="s", num_cores=NC, num_subcores=NS)

_f32 = jnp.float32
_i32 = jnp.int32

_sc_params = pltpu.CompilerParams(needs_layout_passes=False)


@functools.partial(
    pl.kernel,
    out_type=(jax.ShapeDtypeStruct((NC * NP,), _f32),
              jax.ShapeDtypeStruct((E,), _f32),
              jax.ShapeDtypeStruct((E,), _i32)),
    mesh=_mesh,
    compiler_params=_sc_params,
    scratch_types=[
        pltpu.VMEM((N,), _i32),
        pltpu.VMEM((NP,), _f32),
        pltpu.VMEM((EW,), _f32),
        pltpu.VMEM((EW,), _i32),
        pltpu.VMEM((EW,), _i32),
        pltpu.VMEM((EW,), _i32),
        pltpu.VMEM((CH,), _i32),
        pltpu.VMEM((CH,), _i32),
        pltpu.VMEM((CH,), _i32),
        pltpu.VMEM((CH,), _i32),
        pltpu.VMEM((CH, H), _f32),
        pltpu.VMEM((CH, H), _f32),
        pltpu.VMEM((CH, H), _f32),
        pltpu.VMEM((CH, H), _f32),
        pltpu.VMEM((CH,), _f32),
        pltpu.VMEM((NS, RPT), _f32),
        pltpu.VMEM((RPT,), _f32),
        pltpu.VMEM_SHARED((NS, NP), _f32),
        pltpu.SemaphoreType.DMA,
        pltpu.SemaphoreType.DMA,
        pltpu.SemaphoreType.DMA,
        pltpu.SemaphoreType.DMA,
    ],
)
def _k1(ent_hbm, nid_hbm, src_hbm, dst_hbm, maxp_hbm, norm_hbm, csrc_hbm,
        nid_v, maxloc, normloc, sidx, didx, csloc,
        csA, cdA, csB, cdB, srA, drA, srB, drB, nbuf, mt, mred, stg_sh,
        semA1, semA2, semB1, semB2):
    cid = lax.axis_index("c")
    sid = lax.axis_index("s")
    wid = cid * NS + sid
    lane = lax.iota(_i32, 16)
    m15 = lane == 15
    zero16i = jnp.zeros((16,), _i32)

    pltpu.sync_copy(nid_hbm, nid_v)
    pltpu.sync_copy(src_hbm.at[pl.ds(wid * EW, EW)], sidx)
    pltpu.sync_copy(dst_hbm.at[pl.ds(wid * EW, EW)], didx)

    neg = jnp.full((16,), -jnp.inf, _f32)

    @pl.loop(0, NP, step=16)
    def _(i):
        maxloc[pl.ds(i, 16)] = neg

    def compose(cnk, cs, cd):
        coff = cnk * CH

        @pl.loop(0, CH, step=16)
        def _(k):
            s16 = sidx[pl.ds(coff + k, 16)]
            d16 = didx[pl.ds(coff + k, 16)]
            cs16 = plsc.load_gather(nid_v, [s16])
            cs[pl.ds(k, 16)] = cs16
            csloc[pl.ds(coff + k, 16)] = cs16
            cd[pl.ds(k, 16)] = plsc.load_gather(nid_v, [d16])

    def issue(cs, cd, sr, dr, s1, s2):
        pltpu.async_copy(ent_hbm.at[cs], sr, s1)
        pltpu.async_copy(ent_hbm.at[cd], dr, s2)

    def wait(cs, cd, sr, dr, s1, s2):
        pltpu.make_async_copy(ent_hbm.at[cs], sr, s1).wait()
        pltpu.make_async_copy(ent_hbm.at[cd], dr, s2).wait()

    def compute(cnk, sr, dr):
        coff = cnk * CH

        @pl.loop(0, GRP)
        def _(g):
            gbase = g * 16
            for e in range(16):
                r = gbase + e
                acc = sr[r, pl.ds(0, 16)] * dr[r, pl.ds(0, 16)]
                for f in range(1, 8):
                    acc = acc + (sr[r, pl.ds(f * 16, 16)]
                                 * dr[r, pl.ds(f * 16, 16)])
                cum = plsc.cumsum(acc)
                posv = zero16i + r
                plsc.store_scatter(nbuf, [posv], cum, mask=m15)

            n16 = nbuf[pl.ds(gbase, 16)]
            d16 = didx[pl.ds(coff + gbase, 16)]
            cur = plsc.load_gather(maxloc, [d16])
            need = n16 > cur

            def _cond(nd):
                return plsc.all_reduce_population_count(nd)[0] > 0

            def _body(nd):
                plsc.store_scatter(maxloc, [d16], n16, mask=nd)
                c2 = plsc.load_gather(maxloc, [d16])
                return n16 > c2

            lax.while_loop(_cond, _body, need)

        @pl.loop(0, CH, step=16)
        def _(k):
            normloc[pl.ds(coff + k, 16)] = nbuf[pl.ds(k, 16)]

    compose(0, csA, cdA)
    issue(csA, cdA, srA, drA, semA1, semA2)

    @pl.loop(0, NPAIR)
    def _(i):
        c0 = 2 * i
        compose(c0 + 1, csB, cdB)
        issue(csB, cdB, srB, drB, semB1, semB2)
        wait(csA, cdA, srA, drA, semA1, semA2)
        compute(c0, srA, drA)
        compose(c0 + 2, csA, cdA)
        issue(csA, cdA, srA, drA, semA1, semA2)
        wait(csB, cdB, srB, drB, semB1, semB2)
        compute(c0 + 1, srB, drB)

    wait(csA, cdA, srA, drA, semA1, semA2)
    compute(NCHUNK - 1, srA, drA)

    pltpu.sync_copy(normloc, norm_hbm.at[pl.ds(wid * EW, EW)])
    pltpu.sync_copy(csloc, csrc_hbm.at[pl.ds(wid * EW, EW)])

    pltpu.sync_copy(maxloc, stg_sh.at[sid])
    plsc.subcore_barrier()
    pltpu.sync_copy(stg_sh.at[:, pl.ds(sid * RPT, RPT)], mt)

    @pl.loop(0, RPT, step=16)
    def _(k):
        v = mt[0, pl.ds(k, 16)]
        for t in range(1, NS):
            v = jnp.maximum(v, mt[t, pl.ds(k, 16)])
        mred[pl.ds(k, 16)] = v

    pltpu.sync_copy(mred, maxp_hbm.at[pl.ds(cid * NP + sid * RPT, RPT)])


def _k2(maxp):
    def body(x_ref, o_ref):
        o_ref[...] = jnp.max(x_ref[...], axis=0, keepdims=True)

    return pl.pallas_call(
        body,
        out_shape=jax.ShapeDtypeStruct((1, NP), _f32),
    )(maxp)


@functools.partial(
    pl.kernel,
    out_type=(jax.ShapeDtypeStruct((NW * NP,), _f32),
              jax.ShapeDtypeStruct((NC, NP, H), _f32)),
    mesh=_mesh,
    compiler_params=_sc_params,
    scratch_types=[
        pltpu.VMEM((NP,), _f32),
        pltpu.VMEM((NP,), _f32),
        pltpu.VMEM((SUPW,), _i32),
        pltpu.VMEM((SUPW,), _i32),
        pltpu.VMEM((SUPW,), _f32),
        pltpu.VMEM((SUPW,), _i32),
        pltpu.VMEM((SUPW,), _i32),
        pltpu.VMEM((SUPW,), _f32),
        pltpu.VMEM((CH,), _i32),
        pltpu.VMEM((CH,), _i32),
        pltpu.VMEM((CH,), _f32),
        pltpu.VMEM((2560,), _f32),
        pltpu.VMEM((CH, H), _f32),
        pltpu.VMEM((CH, H), _f32),
        pltpu.SemaphoreType.DMA,
        pltpu.SemaphoreType.DMA,
        pltpu.SemaphoreType.DMA,
        pltpu.SemaphoreType.DMA,
        pltpu.SemaphoreType.DMA,
        pltpu.SemaphoreType.DMA,
        pltpu.VMEM_SHARED((NP, H), _f32),
    ],
)
def _k3(ent_hbm, dst_hbm, norm_hbm, csrc_hbm, m_hbm,
        denp_hbm, acc_hbm,
        m_v, denloc, cs0, dd0, nm0, cs1, dd1, nm1, sdA, sdB, wch, mtmp,
        srA, srB, semI0, semI1, semRA, semRB, scatA, scatB, acc_sh):
    cid = lax.axis_index("c")
    sid = lax.axis_index("s")
    wid = cid * NS + sid
    lane = lax.iota(_i32, 16)
    zero16i = jnp.zeros((16,), _i32)
    zv = jnp.zeros((16,), _f32)

    pltpu.sync_copy(m_hbm.at[pl.ds(0, NP)], m_v)
    for q in range(NP // 2560):
        pltpu.sync_copy(m_hbm.at[pl.ds(NP + q * 2560, 2560)], mtmp)

        @pl.loop(0, 2560, step=16)
        def _(k):
            m_v[pl.ds(q * 2560 + k, 16)] = jnp.maximum(
                m_v[pl.ds(q * 2560 + k, 16)], mtmp[pl.ds(k, 16)])

    @pl.loop(0, NP, step=16)
    def _(i):
        denloc[pl.ds(i, 16)] = zv

    @pl.loop(0, CH)
    def _(r):
        for f in range(8):
            srA[r, pl.ds(f * 16, 16)] = zv

    row0 = sid * RPT

    @pl.loop(0, RPT, step=CH)
    def _(r):
        pltpu.sync_copy(srA, acc_sh.at[pl.ds(row0 + r, CH)])

    plsc.subcore_barrier()

    ebase = wid * EW
    sup = (cs0, dd0, nm0, semI0), (cs1, dd1, nm1, semI1)
    rows = (srA, sdA, semRA, scatA), (srB, sdB, semRB, scatB)

    def issue_idx(s, slot):
        cs, dd, nm, sem = sup[slot]
        off = ebase + s * SUPW
        pltpu.async_copy(csrc_hbm.at[pl.ds(off, SUPW)], cs, sem)
        pltpu.async_copy(dst_hbm.at[pl.ds(off, SUPW)], dd, sem)
        pltpu.async_copy(norm_hbm.at[pl.ds(off, SUPW)], nm, sem)

    def wait_idx(s, slot):
        cs, dd, nm, sem = sup[slot]
        off = ebase + s * SUPW
        pltpu.make_async_copy(csrc_hbm.at[pl.ds(off, SUPW)], cs, sem).wait()
        pltpu.make_async_copy(dst_hbm.at[pl.ds(off, SUPW)], dd, sem).wait()
        pltpu.make_async_copy(norm_hbm.at[pl.ds(off, SUPW)], nm, sem).wait()

    def issue_rows(sslot, j, rslot):
        cs = sup[sslot][0]
        sr, _, sem, _ = rows[rslot]
        pltpu.async_copy(ent_hbm.at[cs.at[pl.ds(j * CH, CH)]], sr, sem)

    def wait_rows(sslot, j, rslot):
        cs = sup[sslot][0]
        sr, _, sem, _ = rows[rslot]
        pltpu.make_async_copy(ent_hbm.at[cs.at[pl.ds(j * CH, CH)]],
                              sr, sem).wait()

    def wait_scat(rslot):
        sr, sd, _, ssem = rows[rslot]
        pltpu.make_async_copy(sr, acc_sh.at[sd], ssem).wait()

    def compute(sslot, j, rslot):
        _, dd, nm, _ = sup[sslot]
        sr, sd, _, ssem = rows[rslot]
        base = j * CH

        @pl.loop(0, CH, step=16)
        def _(k):
            d16 = dd[pl.ds(base + k, 16)]
            n16 = nm[pl.ds(base + k, 16)]
            m16 = plsc.load_gather(m_v, [d16])
            w16 = jnp.exp(n16 - m16)
            wch[pl.ds(k, 16)] = w16
            sd[pl.ds(k, 16)] = d16
            plsc.addupdate_scatter(denloc, [d16], w16)

        @pl.loop(0, CH)
        def _(r):
            wspl = plsc.load_gather(wch, [zero16i + r])
            for f in range(8):
                sr[r, pl.ds(f * 16, 16)] = sr[r, pl.ds(f * 16, 16)] * wspl

        pltpu.async_copy(sr, acc_sh.at[sd], ssem, add=True)

    def compute_sync(sslot, j, rslot):
        _, dd, nm, _ = sup[sslot]
        sr, sd, _, _ = rows[rslot]
        base = j * CH

        @pl.loop(0, CH, step=16)
        def _(k):
            d16 = dd[pl.ds(base + k, 16)]
            n16 = nm[pl.ds(base + k, 16)]
            m16 = plsc.load_gather(m_v, [d16])
            w16 = jnp.exp(n16 - m16)
            wch[pl.ds(k, 16)] = w16
            sd[pl.ds(k, 16)] = d16
            for e in range(16):
                plsc.addupdate_scatter(denloc, [d16], w16, mask=(lane == e))

        @pl.loop(0, CH)
        def _(r):
            wspl = plsc.load_gather(wch, [zero16i + r])
            for f in range(8):
                sr[r, pl.ds(f * 16, 16)] = sr[r, pl.ds(f * 16, 16)] * wspl

        pltpu.sync_copy(sr, acc_sh.at[sd], add=True)

    issue_idx(0, 0)
    wait_idx(0, 0)
    issue_rows(0, 0, 0)
    issue_idx(1, 1)

    @pl.loop(0, NSPAIR)
    def _(p):
        for j in range(2 * SUP):
            sslot = 0 if j < SUP else 1
            rslot = j % 2
            nxt = 1 - rslot
            nj = j + 1
            if nj < 2 * SUP:
                if nj == SUP:
                    wait_idx(2 * p + 1, 1)
                if j == 0:
                    @pl.when(p > 0)
                    def _():
                        wait_scat(nxt)
                else:
                    wait_scat(nxt)
                issue_rows(0 if nj < SUP else 1, nj % SUP, nxt)
            wait_rows(sslot, j % SUP, rslot)
            compute(sslot, j % SUP, rslot)
            if j == SUP - 1:
                issue_idx(2 * p + 2, 0)
            if j == 2 * SUP - 1:
                wait_idx(2 * p + 2, 0)
                wait_scat(nxt)
                issue_rows(0, 0, nxt)

                @pl.when(p < NSPAIR - 1)
                def _():
                    issue_idx(2 * p + 3, 1)

    for j in range(SUP):
        rslot = j % 2
        nxt = 1 - rslot
        if j + 1 < SUP:
            wait_scat(nxt)
            issue_rows(0, j + 1, nxt)
        wait_rows(0, j, rslot)
        compute(0, j, rslot)
    wait_scat(1)
    wait_scat(0)

    plsc.subcore_barrier()
    pltpu.sync_copy(denloc, denp_hbm.at[pl.ds(wid * NP, NP)])
    pltpu.sync_copy(acc_sh.at[pl.ds(row0, RPT)],
                    acc_hbm.at[cid, pl.ds(row0, RPT)])


_RB = 1024


def _k4(denp_t, acc0, acc1, w):
    def body(dp_ref, a0_ref, a1_ref, w_ref, o_ref):
        den = jnp.sum(dp_ref[...], axis=1, keepdims=True)
        den = jnp.where(den == 0.0, 1.0, den)
        neigh = (a0_ref[...] + a1_ref[...]) / den
        o_ref[...] = jnp.tanh(
            jnp.dot(neigh, w_ref[...], precision=jax.lax.Precision.HIGHEST))

    return pl.pallas_call(
        body,
        grid=(NP // _RB,),
        in_specs=[
            pl.BlockSpec((_RB, NW), lambda i: (i, 0)),
            pl.BlockSpec((_RB, H), lambda i: (i, 0)),
            pl.BlockSpec((_RB, H), lambda i: (i, 0)),
            pl.BlockSpec((H, H), lambda i: (0, 0)),
        ],
        out_specs=pl.BlockSpec((_RB, H), lambda i: (i, 0)),
        out_shape=jax.ShapeDtypeStruct((NP, H), _f32),
    )(denp_t, acc0, acc1, w)


def kernel(ent_emb, node_id, edge_index, neigh_w):
    nid = node_id.astype(_i32)
    src = edge_index[0].astype(_i32)
    dst = edge_index[1].astype(_i32)
    maxp, norm, csrc = _k1(ent_emb, nid, src, dst)
    denp, acc = _k3(ent_emb, dst, norm, csrc, maxp)
    out = _k4(denp.reshape(NW, NP).T, acc[0], acc[1], neigh_w)
    return out[:N]


# speedup vs baseline: 1.0115x; 1.0115x over previous
import functools

import jax
import jax.numpy as jnp
from jax import lax
from jax.experimental import pallas as pl
from jax.experimental.pallas import tpu as pltpu
from jax.experimental.pallas import tpu_sc as plsc

N = 10000
E = 320000
H = 128
NC, NS = 2, 16
NW = NC * NS
NP = 10240
EW = E // NW
CH = 80
NCHUNK = EW // CH
NPAIR = (NCHUNK - 1) // 2
SUP = 5
SUPW = SUP * CH
NSUP = EW // SUPW
NSPAIR = (NSUP - 1) // 2
GRP = CH // 16
RPT = NP // NS

_mesh = plsc.VectorSubcoreMesh(
    core_axis_name="c", subcore_axis_name="s", num_cores=NC, num_subcores=NS)

_f32 = jnp.float32
_i32 = jnp.int32

_sc_params = pltpu.CompilerParams(needs_layout_passes=False)


@functools.partial(
    pl.kernel,
    out_type=(jax.ShapeDtypeStruct((NW * NP,), _f32),
              jax.ShapeDtypeStruct((E,), _f32),
              jax.ShapeDtypeStruct((E,), _i32)),
    mesh=_mesh,
    compiler_params=_sc_params,
    scratch_types=[
        pltpu.VMEM((N,), _i32),
        pltpu.VMEM((NP,), _f32),
        pltpu.VMEM((EW,), _f32),
        pltpu.VMEM((EW,), _i32),
        pltpu.VMEM((EW,), _i32),
        pltpu.VMEM((EW,), _i32),
        pltpu.VMEM((CH,), _i32),
        pltpu.VMEM((CH,), _i32),
        pltpu.VMEM((CH,), _i32),
        pltpu.VMEM((CH,), _i32),
        pltpu.VMEM((CH, H), _f32),
        pltpu.VMEM((CH, H), _f32),
        pltpu.VMEM((CH, H), _f32),
        pltpu.VMEM((CH, H), _f32),
        pltpu.VMEM((CH,), _f32),
        pltpu.SemaphoreType.DMA,
        pltpu.SemaphoreType.DMA,
        pltpu.SemaphoreType.DMA,
        pltpu.SemaphoreType.DMA,
    ],
)
def _k1(ent_hbm, nid_hbm, src_hbm, dst_hbm, maxp_hbm, norm_hbm, csrc_hbm,
        nid_v, maxloc, normloc, sidx, didx, csloc,
        csA, cdA, csB, cdB, srA, drA, srB, drB, nbuf,
        semA1, semA2, semB1, semB2):
    cid = lax.axis_index("c")
    sid = lax.axis_index("s")
    wid = cid * NS + sid
    lane = lax.iota(_i32, 16)
    m15 = lane == 15
    zero16i = jnp.zeros((16,), _i32)

    pltpu.sync_copy(nid_hbm, nid_v)
    pltpu.sync_copy(src_hbm.at[pl.ds(wid * EW, EW)], sidx)
    pltpu.sync_copy(dst_hbm.at[pl.ds(wid * EW, EW)], didx)

    neg = jnp.full((16,), -jnp.inf, _f32)

    @pl.loop(0, NP, step=16)
    def _(i):
        maxloc[pl.ds(i, 16)] = neg

    def compose(cnk, cs, cd):
        coff = cnk * CH

        @pl.loop(0, CH, step=16)
        def _(k):
            s16 = sidx[pl.ds(coff + k, 16)]
            d16 = didx[pl.ds(coff + k, 16)]
            cs16 = plsc.load_gather(nid_v, [s16])
            cs[pl.ds(k, 16)] = cs16
            csloc[pl.ds(coff + k, 16)] = cs16
            cd[pl.ds(k, 16)] = plsc.load_gather(nid_v, [d16])

    def issue(cs, cd, sr, dr, s1, s2):
        pltpu.async_copy(ent_hbm.at[cs], sr, s1)
        pltpu.async_copy(ent_hbm.at[cd], dr, s2)

    def wait(cs, cd, sr, dr, s1, s2):
        pltpu.make_async_copy(ent_hbm.at[cs], sr, s1).wait()
        pltpu.make_async_copy(ent_hbm.at[cd], dr, s2).wait()

    def compute(cnk, sr, dr):
        coff = cnk * CH

        @pl.loop(0, GRP)
        def _(g):
            gbase = g * 16
            for e in range(16):
                r = gbase + e
                acc = sr[r, pl.ds(0, 16)] * dr[r, pl.ds(0, 16)]
                for f in range(1, 8):
                    acc = acc + (sr[r, pl.ds(f * 16, 16)]
                                 * dr[r, pl.ds(f * 16, 16)])
                cum = plsc.cumsum(acc)
                posv = zero16i + r
                plsc.store_scatter(nbuf, [posv], cum, mask=m15)

            n16 = nbuf[pl.ds(gbase, 16)]
            d16 = didx[pl.ds(coff + gbase, 16)]
            cur = plsc.load_gather(maxloc, [d16])
            need = n16 > cur

            def _cond(nd):
                return plsc.all_reduce_population_count(nd)[0] > 0

            def _body(nd):
                plsc.store_scatter(maxloc, [d16], n16, mask=nd)
                c2 = plsc.load_gather(maxloc, [d16])
                return n16 > c2

            lax.while_loop(_cond, _body, need)

        @pl.loop(0, CH, step=16)
        def _(k):
            normloc[pl.ds(coff + k, 16)] = nbuf[pl.ds(k, 16)]

    compose(0, csA, cdA)
    issue(csA, cdA, srA, drA, semA1, semA2)

    @pl.loop(0, NPAIR)
    def _(i):
        c0 = 2 * i
        compose(c0 + 1, csB, cdB)
        issue(csB, cdB, srB, drB, semB1, semB2)
        wait(csA, cdA, srA, drA, semA1, semA2)
        compute(c0, srA, drA)
        compose(c0 + 2, csA, cdA)
        issue(csA, cdA, srA, drA, semA1, semA2)
        wait(csB, cdB, srB, drB, semB1, semB2)
        compute(c0 + 1, srB, drB)

    wait(csA, cdA, srA, drA, semA1, semA2)
    compute(NCHUNK - 1, srA, drA)

    pltpu.sync_copy(maxloc, maxp_hbm.at[pl.ds(wid * NP, NP)])
    pltpu.sync_copy(normloc, norm_hbm.at[pl.ds(wid * EW, EW)])
    pltpu.sync_copy(csloc, csrc_hbm.at[pl.ds(wid * EW, EW)])


def _k2(maxp):
    def body(x_ref, o_ref):
        o_ref[...] = jnp.max(x_ref[...], axis=0, keepdims=True)

    return pl.pallas_call(
        body,
        out_shape=jax.ShapeDtypeStruct((1, NP), _f32),
    )(maxp)


@functools.partial(
    pl.kernel,
    out_type=(jax.ShapeDtypeStruct((NW * NP,), _f32),
              jax.ShapeDtypeStruct((NC, NP, H), _f32)),
    mesh=_mesh,
    compiler_params=_sc_params,
    scratch_types=[
        pltpu.VMEM((NP,), _f32),
        pltpu.VMEM((NP,), _f32),
        pltpu.VMEM((SUPW,), _i32),
        pltpu.VMEM((SUPW,), _i32),
        pltpu.VMEM((SUPW,), _f32),
        pltpu.VMEM((SUPW,), _i32),
        pltpu.VMEM((SUPW,), _i32),
        pltpu.VMEM((SUPW,), _f32),
        pltpu.VMEM((CH,), _i32),
        pltpu.VMEM((CH,), _i32),
        pltpu.VMEM((CH,), _f32),
        pltpu.VMEM((CH, H), _f32),
        pltpu.VMEM((CH, H), _f32),
        pltpu.SemaphoreType.DMA,
        pltpu.SemaphoreType.DMA,
        pltpu.SemaphoreType.DMA,
        pltpu.SemaphoreType.DMA,
        pltpu.SemaphoreType.DMA,
        pltpu.SemaphoreType.DMA,
        pltpu.VMEM_SHARED((NP, H), _f32),
    ],
)
def _k3(ent_hbm, dst_hbm, norm_hbm, csrc_hbm, m_hbm,
        denp_hbm, acc_hbm,
        m_v, denloc, cs0, dd0, nm0, cs1, dd1, nm1, sdA, sdB, wch,
        srA, srB, semI0, semI1, semRA, semRB, scatA, scatB, acc_sh):
    cid = lax.axis_index("c")
    sid = lax.axis_index("s")
    wid = cid * NS + sid
    lane = lax.iota(_i32, 16)
    zero16i = jnp.zeros((16,), _i32)
    zv = jnp.zeros((16,), _f32)

    pltpu.sync_copy(m_hbm.at[0], m_v)

    @pl.loop(0, NP, step=16)
    def _(i):
        denloc[pl.ds(i, 16)] = zv

    @pl.loop(0, CH)
    def _(r):
        for f in range(8):
            srA[r, pl.ds(f * 16, 16)] = zv

    row0 = sid * RPT

    @pl.loop(0, RPT, step=CH)
    def _(r):
        pltpu.sync_copy(srA, acc_sh.at[pl.ds(row0 + r, CH)])

    plsc.subcore_barrier()

    ebase = wid * EW
    sup = (cs0, dd0, nm0, semI0), (cs1, dd1, nm1, semI1)
    rows = (srA, sdA, semRA, scatA), (srB, sdB, semRB, scatB)

    def issue_idx(s, slot):
        cs, dd, nm, sem = sup[slot]
        off = ebase + s * SUPW
        pltpu.async_copy(csrc_hbm.at[pl.ds(off, SUPW)], cs, sem)
        pltpu.async_copy(dst_hbm.at[pl.ds(off, SUPW)], dd, sem)
        pltpu.async_copy(norm_hbm.at[pl.ds(off, SUPW)], nm, sem)

    def wait_idx(s, slot):
        cs, dd, nm, sem = sup[slot]
        off = ebase + s * SUPW
        pltpu.make_async_copy(csrc_hbm.at[pl.ds(off, SUPW)], cs, sem).wait()
        pltpu.make_async_copy(dst_hbm.at[pl.ds(off, SUPW)], dd, sem).wait()
        pltpu.make_async_copy(norm_hbm.at[pl.ds(off, SUPW)], nm, sem).wait()

    def issue_rows(sslot, j, rslot):
        cs = sup[sslot][0]
        sr, _, sem, _ = rows[rslot]
        pltpu.async_copy(ent_hbm.at[cs.at[pl.ds(j * CH, CH)]], sr, sem)

    def wait_rows(sslot, j, rslot):
        cs = sup[sslot][0]
        sr, _, sem, _ = rows[rslot]
        pltpu.make_async_copy(ent_hbm.at[cs.at[pl.ds(j * CH, CH)]],
                              sr, sem).wait()

    def wait_scat(rslot):
        sr, sd, _, ssem = rows[rslot]
        pltpu.make_async_copy(sr, acc_sh.at[sd], ssem).wait()

    def compute(sslot, j, rslot):
        _, dd, nm, _ = sup[sslot]
        sr, sd, _, ssem = rows[rslot]
        base = j * CH

        @pl.loop(0, CH, step=16)
        def _(k):
            d16 = dd[pl.ds(base + k, 16)]
            n16 = nm[pl.ds(base + k, 16)]
            m16 = plsc.load_gather(m_v, [d16])
            w16 = jnp.exp(n16 - m16)
            wch[pl.ds(k, 16)] = w16
            sd[pl.ds(k, 16)] = d16
            plsc.addupdate_scatter(denloc, [d16], w16)

        @pl.loop(0, CH)
        def _(r):
            wspl = plsc.load_gather(wch, [zero16i + r])
            for f in range(8):
                sr[r, pl.ds(f * 16, 16)] = sr[r, pl.ds(f * 16, 16)] * wspl

        pltpu.async_copy(sr, acc_sh.at[sd], ssem, add=True)

    issue_idx(0, 0)
    wait_idx(0, 0)
    issue_rows(0, 0, 0)
    issue_idx(1, 1)

    @pl.loop(0, NSPAIR)
    def _(p):
        for j in range(2 * SUP):
            sslot = 0 if j < SUP else 1
            rslot = j % 2
            nxt = 1 - rslot
            nj = j + 1
            if nj < 2 * SUP:
                if nj == SUP:
                    wait_idx(2 * p + 1, 1)
                if j == 0:
                    @pl.when(p > 0)
                    def _():
                        wait_scat(nxt)
                else:
                    wait_scat(nxt)
                issue_rows(0 if nj < SUP else 1, nj % SUP, nxt)
            wait_rows(sslot, j % SUP, rslot)
            compute(sslot, j % SUP, rslot)
            if j == SUP - 1:
                issue_idx(2 * p + 2, 0)
            if j == 2 * SUP - 1:
                wait_idx(2 * p + 2, 0)
                wait_scat(nxt)
                issue_rows(0, 0, nxt)

                @pl.when(p < NSPAIR - 1)
                def _():
                    issue_idx(2 * p + 3, 1)

    for j in range(SUP):
        rslot = j % 2
        nxt = 1 - rslot
        if j + 1 < SUP:
            wait_scat(nxt)
            issue_rows(0, j + 1, nxt)
        wait_rows(0, j, rslot)
        compute(0, j, rslot)
    wait_scat(1)
    wait_scat(0)

    plsc.subcore_barrier()
    pltpu.sync_copy(denloc, denp_hbm.at[pl.ds(wid * NP, NP)])
    pltpu.sync_copy(acc_sh.at[pl.ds(row0, RPT)],
                    acc_hbm.at[cid, pl.ds(row0, RPT)])


_RB = 1024


def _k4(denp_t, acc0, acc1, w):
    def body(dp_ref, a0_ref, a1_ref, w_ref, o_ref):
        den = jnp.sum(dp_ref[...], axis=1, keepdims=True)
        den = jnp.where(den == 0.0, 1.0, den)
        neigh = (a0_ref[...] + a1_ref[...]) / den
        o_ref[...] = jnp.tanh(
            jnp.dot(neigh, w_ref[...], precision=jax.lax.Precision.HIGHEST))

    return pl.pallas_call(
        body,
        grid=(NP // _RB,),
        in_specs=[
            pl.BlockSpec((_RB, NW), lambda i: (i, 0)),
            pl.BlockSpec((_RB, H), lambda i: (i, 0)),
            pl.BlockSpec((_RB, H), lambda i: (i, 0)),
            pl.BlockSpec((H, H), lambda i: (0, 0)),
        ],
        out_specs=pl.BlockSpec((_RB, H), lambda i: (i, 0)),
        out_shape=jax.ShapeDtypeStruct((NP, H), _f32),
    )(denp_t, acc0, acc1, w)


def kernel(ent_emb, node_id, edge_index, neigh_w):
    nid = node_id.astype(_i32)
    src = edge_index[0].astype(_i32)
    dst = edge_index[1].astype(_i32)
    maxp, norm, csrc = _k1(ent_emb, nid, src, dst)
    m = _k2(maxp.reshape(NW, NP))
    denp, acc = _k3(ent_emb, dst, norm, csrc, m)
    out = _k4(denp.reshape(NW, NP).T, acc[0], acc[1], neigh_w)
    return out[:N]
